# baseline (speedup 1.0000x reference)
import functools

import jax
import jax.numpy as jnp
from jax import lax
from jax.experimental import pallas as pl
from jax.experimental.pallas import tpu as pltpu
from jax.experimental.pallas import tpu_sc as plsc

NUM_Q = 100000
EMB = 128
PRE = 768
ROW_BLOCK = 1000
NUM_CORES = 2
NUM_SUBCORES = 16
NUM_WORKERS = NUM_CORES * NUM_SUBCORES
CHUNK = 256


def _table_body(id_ref, c_ref, a_ref, wc_ref, wa_ref, bias_ref, type_ref,
                qs_ref, types_ref, out_ref, idx_ref):
    comb = (
        id_ref[...]
        + jnp.dot(c_ref[...].astype(jnp.bfloat16),
                  wc_ref[...].astype(jnp.bfloat16),
                  preferred_element_type=jnp.float32)
        + jnp.dot(a_ref[...].astype(jnp.bfloat16),
                  wa_ref[...].astype(jnp.bfloat16),
                  preferred_element_type=jnp.float32)
        + bias_ref[...]
    )
    out_ref[0, :, :] = comb + type_ref[0:1, :]
    out_ref[1, :, :] = comb + type_ref[1:2, :]
    idx_ref[...] = qs_ref[...] + types_ref[...] * NUM_Q


def _build_expanded_and_indices(id_table, content_table, analysis_table,
                                content_W, analysis_W, bias, type_table,
                                qs2d, types2d):
    grid_n = NUM_Q // ROW_BLOCK
    idx_rows = qs2d.shape[0] // grid_n
    return pl.pallas_call(
        _table_body,
        grid=(grid_n,),
        in_specs=[
            pl.BlockSpec((ROW_BLOCK, EMB), lambda i: (i, 0)),
            pl.BlockSpec((ROW_BLOCK, PRE), lambda i: (i, 0)),
            pl.BlockSpec((ROW_BLOCK, PRE), lambda i: (i, 0)),
            pl.BlockSpec((PRE, EMB), lambda i: (0, 0)),
            pl.BlockSpec((PRE, EMB), lambda i: (0, 0)),
            pl.BlockSpec((1, EMB), lambda i: (0, 0)),
            pl.BlockSpec((2, EMB), lambda i: (0, 0)),
            pl.BlockSpec((idx_rows, 128), lambda i: (i, 0)),
            pl.BlockSpec((idx_rows, 128), lambda i: (i, 0)),
        ],
        out_specs=[
            pl.BlockSpec((2, ROW_BLOCK, EMB), lambda i: (0, i, 0)),
            pl.BlockSpec((idx_rows, 128), lambda i: (i, 0)),
        ],
        out_shape=[
            jax.ShapeDtypeStruct((2, NUM_Q, EMB), jnp.float32),
            jax.ShapeDtypeStruct(qs2d.shape, jnp.int32),
        ],
        compiler_params=pltpu.CompilerParams(dimension_semantics=("parallel",)),
    )(id_table, content_table, analysis_table, content_W, analysis_W, bias,
      type_table, qs2d, types2d)


def _sc_gather(expanded, idx):
    num_idx = idx.shape[0]
    b_per_w = num_idx // NUM_WORKERS
    nchunk = b_per_w // CHUNK
    mesh = plsc.VectorSubcoreMesh(core_axis_name="c", subcore_axis_name="s")

    nround = (nchunk + 2) // 3

    @functools.partial(
        pl.kernel,
        mesh=mesh,
        out_type=jax.ShapeDtypeStruct((num_idx, EMB), jnp.float32),
        scratch_types=[
            pltpu.VMEM((CHUNK,), jnp.int32),
            pltpu.VMEM((CHUNK,), jnp.int32),
            pltpu.VMEM((CHUNK,), jnp.int32),
            pltpu.VMEM((CHUNK, EMB), jnp.float32),
            pltpu.VMEM((CHUNK, EMB), jnp.float32),
            pltpu.VMEM((CHUNK, EMB), jnp.float32),
            pltpu.SemaphoreType.DMA,
            pltpu.SemaphoreType.DMA,
            pltpu.SemaphoreType.DMA,
            pltpu.SemaphoreType.DMA,
            pltpu.SemaphoreType.DMA,
            pltpu.SemaphoreType.DMA,
        ],
    )
    def k(table_hbm, idx_hbm, out_hbm, ix0, ix1, ix2, rw0, rw1, rw2,
          sg0, sg1, sg2, sw0, sw1, sw2):
        idx_v = (ix0, ix1, ix2)
        rows_v = (rw0, rw1, rw2)
        semg = (sg0, sg1, sg2)
        semw = (sw0, sw1, sw2)
        wid = lax.axis_index("s") * NUM_CORES + lax.axis_index("c")
        base0 = wid * b_per_w

        def out_slice(i):
            return out_hbm.at[pl.ds(base0 + i * CHUNK, CHUNK)]

        def substep(i, b):
            @pl.when(i < nchunk)
            def _():
                pltpu.make_async_copy(
                    table_hbm.at[idx_v[b]], rows_v[b], semg[b]).wait()
                pltpu.async_copy(rows_v[b], out_slice(i), semw[b])

            b2 = (b + 2) % 3

            @pl.when(i + 2 < nchunk)
            def _():
                pltpu.sync_copy(
                    idx_hbm.at[pl.ds(base0 + (i + 2) * CHUNK, CHUNK)],
                    idx_v[b2])

                @pl.when(i >= 1)
                def _():
                    pltpu.make_async_copy(
                        rows_v[b2], out_slice(i - 1), semw[b2]).wait()

                pltpu.async_copy(table_hbm.at[idx_v[b2]], rows_v[b2], semg[b2])

        pltpu.sync_copy(idx_hbm.at[pl.ds(base0, CHUNK)], idx_v[0])
        pltpu.async_copy(table_hbm.at[idx_v[0]], rows_v[0], semg[0])
        pltpu.sync_copy(idx_hbm.at[pl.ds(base0 + CHUNK, CHUNK)], idx_v[1])
        pltpu.async_copy(table_hbm.at[idx_v[1]], rows_v[1], semg[1])

        @pl.loop(0, nround)
        def _(r):
            substep(3 * r, 0)
            substep(3 * r + 1, 1)
            substep(3 * r + 2, 2)

        for j in (nchunk - 3, nchunk - 2, nchunk - 1):
            pltpu.make_async_copy(
                rows_v[j % 3], out_slice(j), semw[j % 3]).wait()

    return k(expanded, idx)


def kernel(qs, types, id_table, content_table, content_W, content_b,
           analysis_table, analysis_W, analysis_b, type_table):
    bias = (content_b + analysis_b).reshape(1, EMB)
    batch, seqlen = qs.shape
    qs2d = qs.astype(jnp.int32).reshape(-1, 128)
    types2d = types.astype(jnp.int32).reshape(-1, 128)

    expanded3, idx2d = _build_expanded_and_indices(
        id_table, content_table, analysis_table, content_W, analysis_W, bias,
        type_table, qs2d, types2d,
    )
    expanded = expanded3.reshape(2 * NUM_Q, EMB)
    idx = idx2d.reshape(-1)

    gathered = _sc_gather(expanded, idx)
    return gathered.reshape(batch, seqlen, EMB)


# speedup vs baseline: 1.0085x; 1.0085x over previous
import functools

import jax
import jax.numpy as jnp
from jax import lax
from jax.experimental import pallas as pl
from jax.experimental.pallas import tpu as pltpu
from jax.experimental.pallas import tpu_sc as plsc

NUM_Q = 100000
EMB = 128
PRE = 768
ROW_BLOCK = 2000
NUM_CORES = 2
NUM_SUBCORES = 16
NUM_WORKERS = NUM_CORES * NUM_SUBCORES
CHUNK = 320


def _table_body(id_ref, c_ref, a_ref, wc_ref, wa_ref, bias_ref, type_ref,
                qs_ref, types_ref, out_ref, idx_ref):
    comb = (
        id_ref[...]
        + jnp.dot(c_ref[...].astype(jnp.bfloat16),
                  wc_ref[...].astype(jnp.bfloat16),
                  preferred_element_type=jnp.float32)
        + jnp.dot(a_ref[...].astype(jnp.bfloat16),
                  wa_ref[...].astype(jnp.bfloat16),
                  preferred_element_type=jnp.float32)
        + bias_ref[...]
    )
    out_ref[0, :, :] = comb + type_ref[0:1, :]
    out_ref[1, :, :] = comb + type_ref[1:2, :]
    idx_ref[...] = qs_ref[...] + types_ref[...] * NUM_Q


def _build_expanded_and_indices(id_table, content_table, analysis_table,
                                content_W, analysis_W, bias, type_table,
                                qs2d, types2d):
    grid_n = NUM_Q // ROW_BLOCK
    idx_rows = qs2d.shape[0] // grid_n
    return pl.pallas_call(
        _table_body,
        grid=(grid_n,),
        in_specs=[
            pl.BlockSpec((ROW_BLOCK, EMB), lambda i: (i, 0)),
            pl.BlockSpec((ROW_BLOCK, PRE), lambda i: (i, 0)),
            pl.BlockSpec((ROW_BLOCK, PRE), lambda i: (i, 0)),
            pl.BlockSpec((PRE, EMB), lambda i: (0, 0)),
            pl.BlockSpec((PRE, EMB), lambda i: (0, 0)),
            pl.BlockSpec((1, EMB), lambda i: (0, 0)),
            pl.BlockSpec((2, EMB), lambda i: (0, 0)),
            pl.BlockSpec((idx_rows, 128), lambda i: (i, 0)),
            pl.BlockSpec((idx_rows, 128), lambda i: (i, 0)),
        ],
        out_specs=[
            pl.BlockSpec((2, ROW_BLOCK, EMB), lambda i: (0, i, 0)),
            pl.BlockSpec((idx_rows, 128), lambda i: (i, 0)),
        ],
        out_shape=[
            jax.ShapeDtypeStruct((2, NUM_Q, EMB), jnp.float32),
            jax.ShapeDtypeStruct(qs2d.shape, jnp.int32),
        ],
        compiler_params=pltpu.CompilerParams(dimension_semantics=("parallel",)),
    )(id_table, content_table, analysis_table, content_W, analysis_W, bias,
      type_table, qs2d, types2d)


def _sc_gather(expanded, idx):
    num_idx = idx.shape[0]
    b_per_w = num_idx // NUM_WORKERS
    nchunk = b_per_w // CHUNK
    mesh = plsc.VectorSubcoreMesh(core_axis_name="c", subcore_axis_name="s")

    nround = (nchunk + 2) // 3

    @functools.partial(
        pl.kernel,
        mesh=mesh,
        out_type=jax.ShapeDtypeStruct((num_idx, EMB), jnp.float32),
        scratch_types=[
            pltpu.VMEM((CHUNK,), jnp.int32),
            pltpu.VMEM((CHUNK,), jnp.int32),
            pltpu.VMEM((CHUNK,), jnp.int32),
            pltpu.VMEM((CHUNK, EMB), jnp.float32),
            pltpu.VMEM((CHUNK, EMB), jnp.float32),
            pltpu.VMEM((CHUNK, EMB), jnp.float32),
            pltpu.SemaphoreType.DMA,
            pltpu.SemaphoreType.DMA,
            pltpu.SemaphoreType.DMA,
            pltpu.SemaphoreType.DMA,
            pltpu.SemaphoreType.DMA,
            pltpu.SemaphoreType.DMA,
        ],
    )
    def k(table_hbm, idx_hbm, out_hbm, ix0, ix1, ix2, rw0, rw1, rw2,
          sg0, sg1, sg2, sw0, sw1, sw2):
        idx_v = (ix0, ix1, ix2)
        rows_v = (rw0, rw1, rw2)
        semg = (sg0, sg1, sg2)
        semw = (sw0, sw1, sw2)
        wid = lax.axis_index("s") * NUM_CORES + lax.axis_index("c")
        base0 = wid * b_per_w

        def out_slice(i):
            return out_hbm.at[pl.ds(base0 + i * CHUNK, CHUNK)]

        def substep(i, b):
            @pl.when(i < nchunk)
            def _():
                pltpu.make_async_copy(
                    table_hbm.at[idx_v[b]], rows_v[b], semg[b]).wait()
                pltpu.async_copy(rows_v[b], out_slice(i), semw[b])

            b2 = (b + 2) % 3

            @pl.when(i + 2 < nchunk)
            def _():
                pltpu.sync_copy(
                    idx_hbm.at[pl.ds(base0 + (i + 2) * CHUNK, CHUNK)],
                    idx_v[b2])

                @pl.when(i >= 1)
                def _():
                    pltpu.make_async_copy(
                        rows_v[b2], out_slice(i - 1), semw[b2]).wait()

                pltpu.async_copy(table_hbm.at[idx_v[b2]], rows_v[b2], semg[b2])

        pltpu.sync_copy(idx_hbm.at[pl.ds(base0, CHUNK)], idx_v[0])
        pltpu.async_copy(table_hbm.at[idx_v[0]], rows_v[0], semg[0])
        pltpu.sync_copy(idx_hbm.at[pl.ds(base0 + CHUNK, CHUNK)], idx_v[1])
        pltpu.async_copy(table_hbm.at[idx_v[1]], rows_v[1], semg[1])

        @pl.loop(0, nround)
        def _(r):
            substep(3 * r, 0)
            substep(3 * r + 1, 1)
            substep(3 * r + 2, 2)

        for j in (nchunk - 3, nchunk - 2, nchunk - 1):
            pltpu.make_async_copy(
                rows_v[j % 3], out_slice(j), semw[j % 3]).wait()

    return k(expanded, idx)


def kernel(qs, types, id_table, content_table, content_W, content_b,
           analysis_table, analysis_W, analysis_b, type_table):
    bias = (content_b + analysis_b).reshape(1, EMB)
    batch, seqlen = qs.shape
    qs2d = qs.astype(jnp.int32).reshape(-1, 128)
    types2d = types.astype(jnp.int32).reshape(-1, 128)

    expanded3, idx2d = _build_expanded_and_indices(
        id_table, content_table, analysis_table, content_W, analysis_W, bias,
        type_table, qs2d, types2d,
    )
    expanded = expanded3.reshape(2 * NUM_Q, EMB)
    idx = idx2d.reshape(-1)

    gathered = _sc_gather(expanded, idx)
    return gathered.reshape(batch, seqlen, EMB)


# speedup vs baseline: 1.0096x; 1.0010x over previous
import functools

import jax
import jax.numpy as jnp
from jax import lax
from jax.experimental import pallas as pl
from jax.experimental.pallas import tpu as pltpu
from jax.experimental.pallas import tpu_sc as plsc

NUM_Q = 100000
EMB = 128
PRE = 768
ROW_BLOCK = 2000
NUM_CORES = 2
NUM_SUBCORES = 16
NUM_WORKERS = NUM_CORES * NUM_SUBCORES
CHUNK = 256


def _table_body(id_ref, c_ref, a_ref, wc_ref, wa_ref, bias_ref, type_ref,
                qs_ref, types_ref, out_ref, idx_ref):
    comb = (
        id_ref[...]
        + jnp.dot(c_ref[...].astype(jnp.bfloat16),
                  wc_ref[...].astype(jnp.bfloat16),
                  preferred_element_type=jnp.float32)
        + jnp.dot(a_ref[...].astype(jnp.bfloat16),
                  wa_ref[...].astype(jnp.bfloat16),
                  preferred_element_type=jnp.float32)
        + bias_ref[...]
    )
    out_ref[0, :, :] = comb + type_ref[0:1, :]
    out_ref[1, :, :] = comb + type_ref[1:2, :]
    idx_ref[...] = qs_ref[...] + types_ref[...] * NUM_Q


def _build_expanded_and_indices(id_table, content_table, analysis_table,
                                content_W, analysis_W, bias, type_table,
                                qs2d, types2d):
    grid_n = NUM_Q // ROW_BLOCK
    idx_rows = qs2d.shape[0] // grid_n
    return pl.pallas_call(
        _table_body,
        grid=(grid_n,),
        in_specs=[
            pl.BlockSpec((ROW_BLOCK, EMB), lambda i: (i, 0)),
            pl.BlockSpec((ROW_BLOCK, PRE), lambda i: (i, 0)),
            pl.BlockSpec((ROW_BLOCK, PRE), lambda i: (i, 0)),
            pl.BlockSpec((PRE, EMB), lambda i: (0, 0)),
            pl.BlockSpec((PRE, EMB), lambda i: (0, 0)),
            pl.BlockSpec((1, EMB), lambda i: (0, 0)),
            pl.BlockSpec((2, EMB), lambda i: (0, 0)),
            pl.BlockSpec((idx_rows, 128), lambda i: (i, 0)),
            pl.BlockSpec((idx_rows, 128), lambda i: (i, 0)),
        ],
        out_specs=[
            pl.BlockSpec((2, ROW_BLOCK, EMB), lambda i: (0, i, 0)),
            pl.BlockSpec((idx_rows, 128), lambda i: (i, 0)),
        ],
        out_shape=[
            jax.ShapeDtypeStruct((2, NUM_Q, EMB), jnp.float32),
            jax.ShapeDtypeStruct(qs2d.shape, jnp.int32),
        ],
        compiler_params=pltpu.CompilerParams(dimension_semantics=("parallel",)),
    )(id_table, content_table, analysis_table, content_W, analysis_W, bias,
      type_table, qs2d, types2d)


def _sc_gather(expanded, idx):
    num_idx = idx.shape[0]
    b_per_w = num_idx // NUM_WORKERS
    nchunk = b_per_w // CHUNK
    mesh = plsc.VectorSubcoreMesh(core_axis_name="c", subcore_axis_name="s")

    nround = (nchunk + 2) // 3

    @functools.partial(
        pl.kernel,
        mesh=mesh,
        out_type=jax.ShapeDtypeStruct((num_idx, EMB), jnp.float32),
        scratch_types=[
            pltpu.VMEM((CHUNK,), jnp.int32),
            pltpu.VMEM((CHUNK,), jnp.int32),
            pltpu.VMEM((CHUNK,), jnp.int32),
            pltpu.VMEM((CHUNK, EMB), jnp.float32),
            pltpu.VMEM((CHUNK, EMB), jnp.float32),
            pltpu.VMEM((CHUNK, EMB), jnp.float32),
            pltpu.SemaphoreType.DMA,
            pltpu.SemaphoreType.DMA,
            pltpu.SemaphoreType.DMA,
            pltpu.SemaphoreType.DMA,
            pltpu.SemaphoreType.DMA,
            pltpu.SemaphoreType.DMA,
        ],
    )
    def k(table_hbm, idx_hbm, out_hbm, ix0, ix1, ix2, rw0, rw1, rw2,
          sg0, sg1, sg2, sw0, sw1, sw2):
        idx_v = (ix0, ix1, ix2)
        rows_v = (rw0, rw1, rw2)
        semg = (sg0, sg1, sg2)
        semw = (sw0, sw1, sw2)
        wid = lax.axis_index("s") * NUM_CORES + lax.axis_index("c")
        base0 = wid * b_per_w

        def out_slice(i):
            return out_hbm.at[pl.ds(base0 + i * CHUNK, CHUNK)]

        def substep(i, b):
            @pl.when(i < nchunk)
            def _():
                pltpu.make_async_copy(
                    table_hbm.at[idx_v[b]], rows_v[b], semg[b]).wait()
                pltpu.async_copy(rows_v[b], out_slice(i), semw[b])

            b2 = (b + 2) % 3

            @pl.when(i + 2 < nchunk)
            def _():
                pltpu.sync_copy(
                    idx_hbm.at[pl.ds(base0 + (i + 2) * CHUNK, CHUNK)],
                    idx_v[b2])

                @pl.when(i >= 1)
                def _():
                    pltpu.make_async_copy(
                        rows_v[b2], out_slice(i - 1), semw[b2]).wait()

                pltpu.async_copy(table_hbm.at[idx_v[b2]], rows_v[b2], semg[b2])

        pltpu.sync_copy(idx_hbm.at[pl.ds(base0, CHUNK)], idx_v[0])
        pltpu.async_copy(table_hbm.at[idx_v[0]], rows_v[0], semg[0])
        pltpu.sync_copy(idx_hbm.at[pl.ds(base0 + CHUNK, CHUNK)], idx_v[1])
        pltpu.async_copy(table_hbm.at[idx_v[1]], rows_v[1], semg[1])

        @pl.loop(0, nround)
        def _(r):
            substep(3 * r, 0)
            substep(3 * r + 1, 1)
            substep(3 * r + 2, 2)

        for j in (nchunk - 3, nchunk - 2, nchunk - 1):
            pltpu.make_async_copy(
                rows_v[j % 3], out_slice(j), semw[j % 3]).wait()

    return k(expanded, idx)


def kernel(qs, types, id_table, content_table, content_W, content_b,
           analysis_table, analysis_W, analysis_b, type_table):
    bias = (content_b + analysis_b).reshape(1, EMB)
    batch, seqlen = qs.shape
    qs2d = qs.astype(jnp.int32).reshape(-1, 128)
    types2d = types.astype(jnp.int32).reshape(-1, 128)

    expanded3, idx2d = _build_expanded_and_indices(
        id_table, content_table, analysis_table, content_W, analysis_W, bias,
        type_table, qs2d, types2d,
    )
    expanded = expanded3.reshape(2 * NUM_Q, EMB)
    idx = idx2d.reshape(-1)

    gathered = _sc_gather(expanded, idx)
    return gathered.reshape(batch, seqlen, EMB)
